# baseline (device time: 52238 ns/iter reference)
import jax
import jax.numpy as jnp
from jax import lax
from jax.experimental import pallas as pl
from jax.experimental.pallas import tpu as pltpu

N_DEV = 16
N_FULL_HOPS = 7
N_SEG = 2


def kernel(x):
    m_per, n = x.shape
    seg = m_per // N_SEG
    half = m_per // 2

    def body(x_ref, out_ref, gat_ref, fs_sems, fr_sems, bs_sems, br_sems,
             cp_sems):
        my = lax.axis_index("i")
        q = lax.rem(my, 4)
        z = lax.div(my, 4)
        q_even = lax.rem(q, 2) == 0
        pos = 4 * q + lax.select(q_even, z, 3 - z)

        def ring_at(p):
            p = lax.rem(p + 2 * N_DEV, N_DEV)
            pq = lax.div(p, 4)
            pr = lax.rem(p, 4)
            return pq + 4 * lax.select(lax.rem(pq, 2) == 0, pr, 3 - pr)

        right = ring_at(pos + 1)
        left = ring_at(pos - 1)

        def f_origin(h):
            return ring_at(pos - h)

        def b_origin(h):
            return ring_at(pos + h)

        for s in range(N_SEG):
            gat_ref[pl.ds(my * m_per + s * seg, seg), :] = (
                x_ref[pl.ds(s * seg, seg), :].astype(gat_ref.dtype)
            )

        copies = []

        def out_copy(origin, row0, nrows, ci):
            sl = pl.ds(origin * m_per + row0, nrows)
            cp = pltpu.make_async_copy(gat_ref.at[sl], out_ref.at[sl],
                                       cp_sems.at[ci])
            cp.start()
            copies.append(cp)

        out_copy(my, 0, m_per, 0)

        barrier_sem = pltpu.get_barrier_semaphore()
        for nbr in (left, right):
            pl.semaphore_signal(
                barrier_sem, inc=1,
                device_id=(nbr,), device_id_type=pl.DeviceIdType.MESH,
            )
        pl.semaphore_wait(barrier_sem, 2)

        def seg_copy(origin, s, sems_pair, h, dev):
            sl = pl.ds(origin * m_per + s * seg, seg)
            return pltpu.make_async_remote_copy(
                src_ref=gat_ref.at[sl],
                dst_ref=gat_ref.at[sl],
                send_sem=sems_pair[0].at[h, s],
                recv_sem=sems_pair[1].at[h, s],
                device_id=(dev,),
                device_id_type=pl.DeviceIdType.MESH,
            )

        fwd = (fs_sems, fr_sems)
        bwd = (bs_sems, br_sems)

        def fwd_send(h, s):
            return seg_copy(f_origin(h), s, fwd, h, right)

        def fwd_recv(h, s):
            return seg_copy(f_origin(h + 1), s, fwd, h, left)

        def bwd_send(h, s):
            return seg_copy(b_origin(h), s, bwd, h, left)

        def bwd_recv(h, s):
            return seg_copy(b_origin(h + 1), s, bwd, h, right)

        sends = []

        def start(d):
            d.start()
            sends.append(d)

        for k in range(N_SEG):
            start(fwd_send(0, k))
            start(bwd_send(0, N_SEG - 1 - k))

        for h in range(1, N_FULL_HOPS):
            for k in range(N_SEG):
                fwd_recv(h - 1, k).wait_recv()
                start(fwd_send(h, k))
                bwd_recv(h - 1, N_SEG - 1 - k).wait_recv()
                start(bwd_send(h, N_SEG - 1 - k))
            out_copy(f_origin(h), 0, m_per, h)
            out_copy(b_origin(h), 0, m_per, N_FULL_HOPS + h)

        h6 = N_FULL_HOPS - 1
        h7 = N_FULL_HOPS
        for j in range(N_SEG // 2):
            fwd_recv(h6, j).wait_recv()
            start(fwd_send(h7, j))
            bwd_recv(h6, N_SEG - 1 - j).wait_recv()
            start(bwd_send(h7, N_SEG - 1 - j))

        for s in range(N_SEG // 2, N_SEG):
            fwd_recv(h6, s).wait_recv()
        out_copy(f_origin(h7), 0, m_per, h7)
        for s in reversed(range(N_SEG // 2)):
            bwd_recv(h6, s).wait_recv()
        out_copy(b_origin(h7), 0, m_per, N_FULL_HOPS + h7)
        for j in range(N_SEG // 2):
            fwd_recv(h7, j).wait_recv()
        out_copy(f_origin(h7 + 1), 0, half, 2 * N_FULL_HOPS + 1)
        for j in range(N_SEG // 2):
            bwd_recv(h7, N_SEG - 1 - j).wait_recv()
        out_copy(b_origin(h7 + 1), half, half, 2 * N_FULL_HOPS + 2)

        for d in sends:
            d.wait_send()
        for cp in copies:
            cp.wait()

    return pl.pallas_call(
        body,
        out_shape=jax.ShapeDtypeStruct((N_DEV * m_per, n), jnp.bfloat16),
        in_specs=[pl.BlockSpec(memory_space=pltpu.VMEM)],
        out_specs=pl.BlockSpec(memory_space=pltpu.MemorySpace.HBM),
        scratch_shapes=[
            pltpu.VMEM((N_DEV * m_per, n), jnp.bfloat16),
            pltpu.SemaphoreType.DMA((N_FULL_HOPS + 1, N_SEG)),
            pltpu.SemaphoreType.DMA((N_FULL_HOPS + 1, N_SEG)),
            pltpu.SemaphoreType.DMA((N_FULL_HOPS + 1, N_SEG)),
            pltpu.SemaphoreType.DMA((N_FULL_HOPS + 1, N_SEG)),
            pltpu.SemaphoreType.DMA((2 * N_FULL_HOPS + 3,)),
        ],
        compiler_params=pltpu.CompilerParams(collective_id=0),
    )(x)
